# Initial kernel scaffold; baseline (speedup 1.0000x reference)
import jax
import jax.numpy as jnp
from jax.experimental import pallas as pl


def kernel(x, hyperedge_index, W0, b0, g0, be0, rm0, rv0, W1, b1, g1, be1, rm1, rv1, W2, b2, g2, be2, rm2, rv2, Wc1, bc1, Wc2, bc2):
    raise NotImplementedError("write your pallas kernel here")


# speedup vs baseline: 12.1271x; 12.1271x over previous
import functools

import jax
import jax.numpy as jnp
from jax import lax
from jax.experimental import pallas as pl
from jax.experimental.pallas import tpu as pltpu
from jax.experimental.pallas import tpu_sc as plsc

N = 10000
M = 10000
NNZ = 320000
HID = 64
EPS = 1e-5

NC = 2
NS = 16
NW = NC * NS
PER_W = NNZ // NW
CHUNK = 80
NCHUNK = PER_W // CHUNK
MP = 10240
ROWS_PER_TILE = MP // NS

f32 = jnp.float32


def _mesh():
    return plsc.VectorSubcoreMesh(
        core_axis_name="c", subcore_axis_name="s", num_cores=NC, num_subcores=NS
    )


@functools.cache
def _stage_call():
    @functools.partial(
        pl.kernel,
        out_type=jax.ShapeDtypeStruct((NC, MP, HID), f32),
        mesh=_mesh(),
        compiler_params=pltpu.CompilerParams(use_tc_tiling_on_sc=False),
        scratch_types=[
            pltpu.VMEM((NCHUNK, CHUNK), jnp.int32),
            pltpu.VMEM((NCHUNK, CHUNK), jnp.int32),
            pltpu.VMEM((CHUNK, HID), f32),
            pltpu.VMEM_SHARED((MP, HID), f32),
        ],
    )
    def stage(table_hbm, gidx_hbm, sidx_hbm, zeros_hbm, out_hbm,
              gix_v, six_v, rows_v, acc):
        cid = lax.axis_index("c")
        sid = lax.axis_index("s")
        wid = cid * NS + sid
        base = sid * ROWS_PER_TILE
        pltpu.sync_copy(zeros_hbm.at[pl.ds(base, ROWS_PER_TILE)],
                        acc.at[pl.ds(base, ROWS_PER_TILE)])
        pltpu.sync_copy(gidx_hbm.at[wid], gix_v)
        pltpu.sync_copy(sidx_hbm.at[wid], six_v)
        plsc.subcore_barrier()

        @pl.loop(0, NCHUNK)
        def _(ci):
            pltpu.sync_copy(table_hbm.at[gix_v.at[ci]], rows_v)
            pltpu.sync_copy(rows_v, acc.at[six_v.at[ci]], add=True)

        plsc.subcore_barrier()
        pltpu.sync_copy(acc.at[pl.ds(base, ROWS_PER_TILE)],
                        out_hbm.at[cid, pl.ds(base, ROWS_PER_TILE)])

    return stage


@functools.cache
def _hist_call():
    @functools.partial(
        pl.kernel,
        out_type=jax.ShapeDtypeStruct((NC, 2, MP, 16), f32),
        mesh=_mesh(),
        compiler_params=pltpu.CompilerParams(use_tc_tiling_on_sc=False),
        scratch_types=[
            pltpu.VMEM((NCHUNK, CHUNK), jnp.int32),
            pltpu.VMEM((NCHUNK, CHUNK), jnp.int32),
            pltpu.VMEM((CHUNK, 16), f32),
            pltpu.VMEM_SHARED((MP, 16), f32),
            pltpu.VMEM_SHARED((MP, 16), f32),
        ],
    )
    def hist(src_hbm, he_hbm, zeros16_hbm, ones_hbm, out_hbm,
             src_v, he_v, ones_v, dacc, eacc):
        cid = lax.axis_index("c")
        sid = lax.axis_index("s")
        wid = cid * NS + sid
        base = sid * ROWS_PER_TILE
        pltpu.sync_copy(zeros16_hbm.at[pl.ds(base, ROWS_PER_TILE)],
                        dacc.at[pl.ds(base, ROWS_PER_TILE)])
        pltpu.sync_copy(zeros16_hbm.at[pl.ds(base, ROWS_PER_TILE)],
                        eacc.at[pl.ds(base, ROWS_PER_TILE)])
        pltpu.sync_copy(ones_hbm, ones_v)
        pltpu.sync_copy(src_hbm.at[wid], src_v)
        pltpu.sync_copy(he_hbm.at[wid], he_v)
        plsc.subcore_barrier()

        @pl.loop(0, NCHUNK)
        def _(ci):
            pltpu.sync_copy(ones_v, dacc.at[src_v.at[ci]], add=True)
            pltpu.sync_copy(ones_v, eacc.at[he_v.at[ci]], add=True)

        plsc.subcore_barrier()
        pltpu.sync_copy(dacc.at[pl.ds(base, ROWS_PER_TILE)],
                        out_hbm.at[cid, 0, pl.ds(base, ROWS_PER_TILE)])
        pltpu.sync_copy(eacc.at[pl.ds(base, ROWS_PER_TILE)],
                        out_hbm.at[cid, 1, pl.ds(base, ROWS_PER_TILE)])

    return hist


BN_ROWS = 1000


def _tc_matmul0(x, w):
    def body(x_ref, w_ref, o_ref):
        o_ref[...] = jnp.dot(x_ref[...], w_ref[...],
                             preferred_element_type=f32)

    d_in = x.shape[1]
    return pl.pallas_call(
        body,
        grid=(N // BN_ROWS,),
        in_specs=[
            pl.BlockSpec((BN_ROWS, d_in), lambda i: (i, 0)),
            pl.BlockSpec((d_in, HID), lambda i: (0, 0)),
        ],
        out_specs=pl.BlockSpec((BN_ROWS, HID), lambda i: (i, 0)),
        out_shape=jax.ShapeDtypeStruct((N, HID), f32),
    )(x, w)


def _tc_combine_e(pe, counts):
    def body(pe_ref, cnt_ref, o_ref):
        s = pe_ref[0] + pe_ref[1]
        edeg = cnt_ref[0, 0, :, 0:1] + cnt_ref[1, 0, :, 0:1]
        binv = jnp.where(edeg > 0, 1.0 / edeg, 0.0)
        o_ref[...] = s * binv

    return pl.pallas_call(
        body,
        grid=(M // BN_ROWS,),
        in_specs=[
            pl.BlockSpec((NC, BN_ROWS, HID), lambda i: (0, i, 0)),
            pl.BlockSpec((NC, 1, BN_ROWS, 16), lambda i: (0, 1, i, 0)),
        ],
        out_specs=pl.BlockSpec((BN_ROWS, HID), lambda i: (i, 0)),
        out_shape=jax.ShapeDtypeStruct((M, HID), f32),
    )(pe, counts)


def _tc_epilogue_matmul(pn, counts, b, g, be, rm, rv, w):
    def body(pn_ref, cnt_ref, b_ref, g_ref, be_ref, rm_ref, rv_ref, w_ref,
             o_ref):
        s = pn_ref[0] + pn_ref[1]
        deg = cnt_ref[0, 0, :, 0:1] + cnt_ref[1, 0, :, 0:1]
        dinv = jnp.where(deg > 0, 1.0 / deg, 0.0)
        scale = g_ref[...] * lax.rsqrt(rv_ref[...] + EPS)
        shift = (b_ref[...] - rm_ref[...]) * scale + be_ref[...]
        h = jnp.maximum(s * dinv * scale + shift, 0.0)
        o_ref[...] = jnp.dot(h, w_ref[...], preferred_element_type=f32)

    vec = lambda: pl.BlockSpec((1, HID), lambda i: (0, 0))
    return pl.pallas_call(
        body,
        grid=(N // BN_ROWS,),
        in_specs=[
            pl.BlockSpec((NC, BN_ROWS, HID), lambda i: (0, i, 0)),
            pl.BlockSpec((NC, 1, BN_ROWS, 16), lambda i: (0, 0, i, 0)),
            vec(), vec(), vec(), vec(), vec(),
            pl.BlockSpec((HID, HID), lambda i: (0, 0)),
        ],
        out_specs=pl.BlockSpec((BN_ROWS, HID), lambda i: (i, 0)),
        out_shape=jax.ShapeDtypeStruct((N, HID), f32),
    )(pn, counts, b.reshape(1, HID), g.reshape(1, HID), be.reshape(1, HID),
      rm.reshape(1, HID), rv.reshape(1, HID), w)


def _tc_epilogue_head(pn, counts, b, g, be, rm, rv, wc1, bc1, wc2, bc2):
    h1 = wc1.shape[1]
    ncls = wc2.shape[1]

    def body(pn_ref, cnt_ref, b_ref, g_ref, be_ref, rm_ref, rv_ref,
             wc1_ref, bc1_ref, wc2_ref, bc2_ref, o_ref):
        s = pn_ref[0] + pn_ref[1]
        deg = cnt_ref[0, 0, :, 0:1] + cnt_ref[1, 0, :, 0:1]
        dinv = jnp.where(deg > 0, 1.0 / deg, 0.0)
        scale = g_ref[...] * lax.rsqrt(rv_ref[...] + EPS)
        shift = (b_ref[...] - rm_ref[...]) * scale + be_ref[...]
        h = jnp.maximum(s * dinv * scale + shift, 0.0)
        t = jnp.maximum(
            jnp.dot(h, wc1_ref[...], preferred_element_type=f32)
            + bc1_ref[...], 0.0)
        o_ref[...] = (jnp.dot(t, wc2_ref[...], preferred_element_type=f32)
                      + bc2_ref[...])

    vec = lambda: pl.BlockSpec((1, HID), lambda i: (0, 0))
    return pl.pallas_call(
        body,
        grid=(N // BN_ROWS,),
        in_specs=[
            pl.BlockSpec((NC, BN_ROWS, HID), lambda i: (0, i, 0)),
            pl.BlockSpec((NC, 1, BN_ROWS, 16), lambda i: (0, 0, i, 0)),
            vec(), vec(), vec(), vec(), vec(),
            pl.BlockSpec((HID, h1), lambda i: (0, 0)),
            pl.BlockSpec((1, h1), lambda i: (0, 0)),
            pl.BlockSpec((h1, ncls), lambda i: (0, 0)),
            pl.BlockSpec((1, ncls), lambda i: (0, 0)),
        ],
        out_specs=pl.BlockSpec((BN_ROWS, ncls), lambda i: (i, 0)),
        out_shape=jax.ShapeDtypeStruct((N, ncls), f32),
    )(pn, counts, b.reshape(1, HID), g.reshape(1, HID), be.reshape(1, HID),
      rm.reshape(1, HID), rv.reshape(1, HID), wc1, bc1.reshape(1, h1),
      wc2, bc2.reshape(1, ncls))


def kernel(x, hyperedge_index, W0, b0, g0, be0, rm0, rv0,
           W1, b1, g1, be1, rm1, rv1, W2, b2, g2, be2, rm2, rv2,
           Wc1, bc1, Wc2, bc2):
    src = hyperedge_index[0].reshape(NW, NCHUNK, CHUNK)
    he = hyperedge_index[1].reshape(NW, NCHUNK, CHUNK)
    zeros64 = jnp.zeros((MP, HID), f32)
    zeros16 = jnp.zeros((MP, 16), f32)
    ones16 = jnp.ones((CHUNK, 16), f32)

    counts = _hist_call()(src, he, zeros16, ones16)
    stage = _stage_call()

    params = [(b0, g0, be0, rm0, rv0),
              (b1, g1, be1, rm1, rv1),
              (b2, g2, be2, rm2, rv2)]
    next_w = [W1, W2]
    xw = _tc_matmul0(x, W0)
    for li, (b, g, be, rm, rv) in enumerate(params):
        pe = stage(xw, src, he, zeros64)
        out_e = _tc_combine_e(pe, counts)
        pn = stage(out_e, he, src, zeros64)
        if li < 2:
            xw = _tc_epilogue_matmul(pn, counts, b, g, be, rm, rv, next_w[li])
        else:
            return _tc_epilogue_head(pn, counts, b, g, be, rm, rv,
                                     Wc1, bc1, Wc2, bc2)


# speedup vs baseline: 18.8817x; 1.5570x over previous
import functools

import jax
import jax.numpy as jnp
from jax import lax
from jax.experimental import pallas as pl
from jax.experimental.pallas import tpu as pltpu
from jax.experimental.pallas import tpu_sc as plsc

N = 10000
M = 10000
NNZ = 320000
HID = 64
EPS = 1e-5

NC = 2
NS = 16
NW = NC * NS
PER_W = NNZ // NW
CHUNK = 80
NCHUNK = PER_W // CHUNK
MP = 10240
ROWS_PER_TILE = MP // NS

f32 = jnp.float32


def _mesh():
    return plsc.VectorSubcoreMesh(
        core_axis_name="c", subcore_axis_name="s", num_cores=NC, num_subcores=NS
    )


@functools.cache
def _stage_call():
    @functools.partial(
        pl.kernel,
        out_type=jax.ShapeDtypeStruct((NC, MP, HID), f32),
        mesh=_mesh(),
        compiler_params=pltpu.CompilerParams(use_tc_tiling_on_sc=False),
        scratch_types=[
            pltpu.VMEM((NCHUNK, CHUNK), jnp.int32),
            pltpu.VMEM((NCHUNK, CHUNK), jnp.int32),
            pltpu.VMEM((CHUNK, HID), f32),
            pltpu.VMEM((CHUNK, HID), f32),
            pltpu.VMEM_SHARED((MP, HID), f32),
            pltpu.SemaphoreType.DMA,
            pltpu.SemaphoreType.DMA,
        ],
    )
    def stage(table_hbm, gidx_hbm, sidx_hbm, zeros_hbm, out_hbm,
              gix_v, six_v, rows_a, rows_b, acc, sem_a, sem_b):
        cid = lax.axis_index("c")
        sid = lax.axis_index("s")
        wid = cid * NS + sid
        base = sid * ROWS_PER_TILE
        pltpu.sync_copy(zeros_hbm.at[pl.ds(base, ROWS_PER_TILE)],
                        acc.at[pl.ds(base, ROWS_PER_TILE)])
        pltpu.sync_copy(gidx_hbm.at[wid], gix_v)
        pltpu.sync_copy(sidx_hbm.at[wid], six_v)
        plsc.subcore_barrier()

        def start_gather(ci, buf, sem):
            pltpu.async_copy(table_hbm.at[gix_v.at[ci]], buf, sem)

        def wait_gather(buf, sem):
            pltpu.make_async_copy(table_hbm.at[gix_v.at[0]], buf, sem).wait()

        start_gather(0, rows_a, sem_a)

        @pl.loop(0, NCHUNK - 1, step=2)
        def _(ci):
            start_gather(ci + 1, rows_b, sem_b)
            wait_gather(rows_a, sem_a)
            pltpu.sync_copy(rows_a, acc.at[six_v.at[ci]], add=True)
            start_gather(ci + 2, rows_a, sem_a)
            wait_gather(rows_b, sem_b)
            pltpu.sync_copy(rows_b, acc.at[six_v.at[ci + 1]], add=True)

        wait_gather(rows_a, sem_a)
        pltpu.sync_copy(rows_a, acc.at[six_v.at[NCHUNK - 1]], add=True)

        plsc.subcore_barrier()
        pltpu.sync_copy(acc.at[pl.ds(base, ROWS_PER_TILE)],
                        out_hbm.at[cid, pl.ds(base, ROWS_PER_TILE)])

    return stage


@functools.cache
def _hist_call():
    @functools.partial(
        pl.kernel,
        out_type=jax.ShapeDtypeStruct((NC, 2, MP, 16), f32),
        mesh=_mesh(),
        compiler_params=pltpu.CompilerParams(use_tc_tiling_on_sc=False),
        scratch_types=[
            pltpu.VMEM((NCHUNK, CHUNK), jnp.int32),
            pltpu.VMEM((NCHUNK, CHUNK), jnp.int32),
            pltpu.VMEM((CHUNK, 16), f32),
            pltpu.VMEM_SHARED((MP, 16), f32),
            pltpu.VMEM_SHARED((MP, 16), f32),
            pltpu.SemaphoreType.DMA,
            pltpu.SemaphoreType.DMA,
        ],
    )
    def hist(src_hbm, he_hbm, zeros16_hbm, ones_hbm, out_hbm,
             src_v, he_v, ones_v, dacc, eacc, sem_d, sem_e):
        cid = lax.axis_index("c")
        sid = lax.axis_index("s")
        wid = cid * NS + sid
        base = sid * ROWS_PER_TILE
        pltpu.sync_copy(zeros16_hbm.at[pl.ds(base, ROWS_PER_TILE)],
                        dacc.at[pl.ds(base, ROWS_PER_TILE)])
        pltpu.sync_copy(zeros16_hbm.at[pl.ds(base, ROWS_PER_TILE)],
                        eacc.at[pl.ds(base, ROWS_PER_TILE)])
        pltpu.sync_copy(ones_hbm, ones_v)
        pltpu.sync_copy(src_hbm.at[wid], src_v)
        pltpu.sync_copy(he_hbm.at[wid], he_v)
        plsc.subcore_barrier()

        K = 5

        @pl.loop(0, NCHUNK, step=K)
        def _(ci):
            for j in range(K):
                pltpu.async_copy(ones_v, dacc.at[src_v.at[ci + j]], sem_d,
                                 add=True)
                pltpu.async_copy(ones_v, eacc.at[he_v.at[ci + j]], sem_e,
                                 add=True)
            for j in range(K):
                pltpu.make_async_copy(ones_v, dacc.at[src_v.at[ci]],
                                      sem_d).wait()
                pltpu.make_async_copy(ones_v, eacc.at[he_v.at[ci]],
                                      sem_e).wait()

        plsc.subcore_barrier()
        pltpu.sync_copy(dacc.at[pl.ds(base, ROWS_PER_TILE)],
                        out_hbm.at[cid, 0, pl.ds(base, ROWS_PER_TILE)])
        pltpu.sync_copy(eacc.at[pl.ds(base, ROWS_PER_TILE)],
                        out_hbm.at[cid, 1, pl.ds(base, ROWS_PER_TILE)])

    return hist


BN_ROWS = 1000


def _tc_matmul0(x, w):
    def body(x_ref, w_ref, o_ref):
        o_ref[...] = jnp.dot(x_ref[...], w_ref[...],
                             preferred_element_type=f32)

    d_in = x.shape[1]
    return pl.pallas_call(
        body,
        grid=(N // BN_ROWS,),
        in_specs=[
            pl.BlockSpec((BN_ROWS, d_in), lambda i: (i, 0)),
            pl.BlockSpec((d_in, HID), lambda i: (0, 0)),
        ],
        out_specs=pl.BlockSpec((BN_ROWS, HID), lambda i: (i, 0)),
        out_shape=jax.ShapeDtypeStruct((N, HID), f32),
    )(x, w)


def _tc_combine_e(pe, counts):
    def body(pe_ref, cnt_ref, o_ref):
        s = pe_ref[0] + pe_ref[1]
        edeg = cnt_ref[0, 0, :, 0:1] + cnt_ref[1, 0, :, 0:1]
        binv = jnp.where(edeg > 0, 1.0 / edeg, 0.0)
        o_ref[...] = s * binv

    return pl.pallas_call(
        body,
        grid=(M // BN_ROWS,),
        in_specs=[
            pl.BlockSpec((NC, BN_ROWS, HID), lambda i: (0, i, 0)),
            pl.BlockSpec((NC, 1, BN_ROWS, 16), lambda i: (0, 1, i, 0)),
        ],
        out_specs=pl.BlockSpec((BN_ROWS, HID), lambda i: (i, 0)),
        out_shape=jax.ShapeDtypeStruct((M, HID), f32),
    )(pe, counts)


def _tc_epilogue_matmul(pn, counts, b, g, be, rm, rv, w):
    def body(pn_ref, cnt_ref, b_ref, g_ref, be_ref, rm_ref, rv_ref, w_ref,
             o_ref):
        s = pn_ref[0] + pn_ref[1]
        deg = cnt_ref[0, 0, :, 0:1] + cnt_ref[1, 0, :, 0:1]
        dinv = jnp.where(deg > 0, 1.0 / deg, 0.0)
        scale = g_ref[...] * lax.rsqrt(rv_ref[...] + EPS)
        shift = (b_ref[...] - rm_ref[...]) * scale + be_ref[...]
        h = jnp.maximum(s * dinv * scale + shift, 0.0)
        o_ref[...] = jnp.dot(h, w_ref[...], preferred_element_type=f32)

    vec = lambda: pl.BlockSpec((1, HID), lambda i: (0, 0))
    return pl.pallas_call(
        body,
        grid=(N // BN_ROWS,),
        in_specs=[
            pl.BlockSpec((NC, BN_ROWS, HID), lambda i: (0, i, 0)),
            pl.BlockSpec((NC, 1, BN_ROWS, 16), lambda i: (0, 0, i, 0)),
            vec(), vec(), vec(), vec(), vec(),
            pl.BlockSpec((HID, HID), lambda i: (0, 0)),
        ],
        out_specs=pl.BlockSpec((BN_ROWS, HID), lambda i: (i, 0)),
        out_shape=jax.ShapeDtypeStruct((N, HID), f32),
    )(pn, counts, b.reshape(1, HID), g.reshape(1, HID), be.reshape(1, HID),
      rm.reshape(1, HID), rv.reshape(1, HID), w)


def _tc_epilogue_head(pn, counts, b, g, be, rm, rv, wc1, bc1, wc2, bc2):
    h1 = wc1.shape[1]
    ncls = wc2.shape[1]

    def body(pn_ref, cnt_ref, b_ref, g_ref, be_ref, rm_ref, rv_ref,
             wc1_ref, bc1_ref, wc2_ref, bc2_ref, o_ref):
        s = pn_ref[0] + pn_ref[1]
        deg = cnt_ref[0, 0, :, 0:1] + cnt_ref[1, 0, :, 0:1]
        dinv = jnp.where(deg > 0, 1.0 / deg, 0.0)
        scale = g_ref[...] * lax.rsqrt(rv_ref[...] + EPS)
        shift = (b_ref[...] - rm_ref[...]) * scale + be_ref[...]
        h = jnp.maximum(s * dinv * scale + shift, 0.0)
        t = jnp.maximum(
            jnp.dot(h, wc1_ref[...], preferred_element_type=f32)
            + bc1_ref[...], 0.0)
        o_ref[...] = (jnp.dot(t, wc2_ref[...], preferred_element_type=f32)
                      + bc2_ref[...])

    vec = lambda: pl.BlockSpec((1, HID), lambda i: (0, 0))
    return pl.pallas_call(
        body,
        grid=(N // BN_ROWS,),
        in_specs=[
            pl.BlockSpec((NC, BN_ROWS, HID), lambda i: (0, i, 0)),
            pl.BlockSpec((NC, 1, BN_ROWS, 16), lambda i: (0, 0, i, 0)),
            vec(), vec(), vec(), vec(), vec(),
            pl.BlockSpec((HID, h1), lambda i: (0, 0)),
            pl.BlockSpec((1, h1), lambda i: (0, 0)),
            pl.BlockSpec((h1, ncls), lambda i: (0, 0)),
            pl.BlockSpec((1, ncls), lambda i: (0, 0)),
        ],
        out_specs=pl.BlockSpec((BN_ROWS, ncls), lambda i: (i, 0)),
        out_shape=jax.ShapeDtypeStruct((N, ncls), f32),
    )(pn, counts, b.reshape(1, HID), g.reshape(1, HID), be.reshape(1, HID),
      rm.reshape(1, HID), rv.reshape(1, HID), wc1, bc1.reshape(1, h1),
      wc2, bc2.reshape(1, ncls))


def kernel(x, hyperedge_index, W0, b0, g0, be0, rm0, rv0,
           W1, b1, g1, be1, rm1, rv1, W2, b2, g2, be2, rm2, rv2,
           Wc1, bc1, Wc2, bc2):
    src = hyperedge_index[0].reshape(NW, NCHUNK, CHUNK)
    he = hyperedge_index[1].reshape(NW, NCHUNK, CHUNK)
    zeros64 = jnp.zeros((MP, HID), f32)
    zeros16 = jnp.zeros((MP, 16), f32)
    ones16 = jnp.ones((CHUNK, 16), f32)

    counts = _hist_call()(src, he, zeros16, ones16)
    stage = _stage_call()

    params = [(b0, g0, be0, rm0, rv0),
              (b1, g1, be1, rm1, rv1),
              (b2, g2, be2, rm2, rv2)]
    next_w = [W1, W2]
    xw = _tc_matmul0(x, W0)
    for li, (b, g, be, rm, rv) in enumerate(params):
        pe = stage(xw, src, he, zeros64)
        out_e = _tc_combine_e(pe, counts)
        pn = stage(out_e, he, src, zeros64)
        if li < 2:
            xw = _tc_epilogue_matmul(pn, counts, b, g, be, rm, rv, next_w[li])
        else:
            return _tc_epilogue_head(pn, counts, b, g, be, rm, rv,
                                     Wc1, bc1, Wc2, bc2)


# speedup vs baseline: 21.6214x; 1.1451x over previous
import functools

import jax
import jax.numpy as jnp
from jax import lax
from jax.experimental import pallas as pl
from jax.experimental.pallas import tpu as pltpu
from jax.experimental.pallas import tpu_sc as plsc

N = 10000
M = 10000
NNZ = 320000
HID = 64
EPS = 1e-5

NC = 2
NS = 16
NW = NC * NS
PER_W = NNZ // NW
CHUNK = 128
PER_W_PAD = 10240
PAD = PER_W_PAD - PER_W
NCHUNK = PER_W_PAD // CHUNK
NBUF = 4
MP = 10240
ROWS_PER_TILE = MP // NS

f32 = jnp.float32


def _mesh():
    return plsc.VectorSubcoreMesh(
        core_axis_name="c", subcore_axis_name="s", num_cores=NC, num_subcores=NS
    )


@functools.cache
def _stage_call():
    @functools.partial(
        pl.kernel,
        out_type=jax.ShapeDtypeStruct((NC, MP, HID), f32),
        mesh=_mesh(),
        compiler_params=pltpu.CompilerParams(use_tc_tiling_on_sc=False),
        scratch_types=[
            pltpu.VMEM((NCHUNK, CHUNK), jnp.int32),
            pltpu.VMEM((NCHUNK, CHUNK), jnp.int32),
        ] + [pltpu.VMEM((CHUNK, HID), f32)] * NBUF
          + [pltpu.VMEM_SHARED((MP, HID), f32)]
          + [pltpu.SemaphoreType.DMA] * (2 * NBUF),
    )
    def stage(table_hbm, gidx_hbm, sidx_hbm, zeros_hbm, out_hbm,
              gix_v, six_v, *rest):
        bufs = rest[:NBUF]
        acc = rest[NBUF]
        gsems = rest[NBUF + 1:NBUF + 1 + NBUF]
        ssems = rest[NBUF + 1 + NBUF:]
        cid = lax.axis_index("c")
        sid = lax.axis_index("s")
        wid = cid * NS + sid
        base = sid * ROWS_PER_TILE
        pltpu.sync_copy(zeros_hbm.at[pl.ds(base, ROWS_PER_TILE)],
                        acc.at[pl.ds(base, ROWS_PER_TILE)])
        pltpu.sync_copy(gidx_hbm.at[wid], gix_v)
        pltpu.sync_copy(sidx_hbm.at[wid], six_v)
        plsc.subcore_barrier()

        def start_gather(ci, b):
            pltpu.async_copy(table_hbm.at[gix_v.at[ci]], bufs[b], gsems[b])

        def wait_gather(b):
            pltpu.make_async_copy(table_hbm.at[gix_v.at[0]], bufs[b],
                                  gsems[b]).wait()

        def start_scatter(ci, b):
            pltpu.async_copy(bufs[b], acc.at[six_v.at[ci]], ssems[b],
                             add=True)

        def wait_scatter(b):
            pltpu.make_async_copy(bufs[b], acc.at[six_v.at[0]],
                                  ssems[b]).wait()

        for b in range(NBUF):
            start_gather(b, b)

        @pl.loop(0, NCHUNK, step=NBUF)
        def _(ci):
            for b in range(NBUF):
                wait_gather(b)
                start_scatter(ci + b, b)
            for b in range(NBUF):
                wait_scatter(b)
                start_gather(jnp.minimum(ci + NBUF + b, NCHUNK - 1), b)

        for b in range(NBUF):
            wait_gather(b)

        plsc.subcore_barrier()
        pltpu.sync_copy(acc.at[pl.ds(base, ROWS_PER_TILE)],
                        out_hbm.at[cid, pl.ds(base, ROWS_PER_TILE)])

    return stage


@functools.cache
def _hist_call():
    @functools.partial(
        pl.kernel,
        out_type=jax.ShapeDtypeStruct((NC, 2, MP, 16), f32),
        mesh=_mesh(),
        compiler_params=pltpu.CompilerParams(use_tc_tiling_on_sc=False),
        scratch_types=[
            pltpu.VMEM((NCHUNK, CHUNK), jnp.int32),
            pltpu.VMEM((NCHUNK, CHUNK), jnp.int32),
            pltpu.VMEM((CHUNK, 16), f32),
            pltpu.VMEM_SHARED((MP, 16), f32),
            pltpu.VMEM_SHARED((MP, 16), f32),
            pltpu.SemaphoreType.DMA,
            pltpu.SemaphoreType.DMA,
        ],
    )
    def hist(src_hbm, he_hbm, zeros16_hbm, ones_hbm, out_hbm,
             src_v, he_v, ones_v, dacc, eacc, sem_d, sem_e):
        cid = lax.axis_index("c")
        sid = lax.axis_index("s")
        wid = cid * NS + sid
        base = sid * ROWS_PER_TILE
        pltpu.sync_copy(zeros16_hbm.at[pl.ds(base, ROWS_PER_TILE)],
                        dacc.at[pl.ds(base, ROWS_PER_TILE)])
        pltpu.sync_copy(zeros16_hbm.at[pl.ds(base, ROWS_PER_TILE)],
                        eacc.at[pl.ds(base, ROWS_PER_TILE)])
        pltpu.sync_copy(ones_hbm, ones_v)
        pltpu.sync_copy(src_hbm.at[wid], src_v)
        pltpu.sync_copy(he_hbm.at[wid], he_v)
        plsc.subcore_barrier()

        K = 5

        @pl.loop(0, NCHUNK, step=K)
        def _(ci):
            for j in range(K):
                pltpu.async_copy(ones_v, dacc.at[src_v.at[ci + j]], sem_d,
                                 add=True)
                pltpu.async_copy(ones_v, eacc.at[he_v.at[ci + j]], sem_e,
                                 add=True)
            for j in range(K):
                pltpu.make_async_copy(ones_v, dacc.at[src_v.at[ci]],
                                      sem_d).wait()
                pltpu.make_async_copy(ones_v, eacc.at[he_v.at[ci]],
                                      sem_e).wait()

        plsc.subcore_barrier()
        pltpu.sync_copy(dacc.at[pl.ds(base, ROWS_PER_TILE)],
                        out_hbm.at[cid, 0, pl.ds(base, ROWS_PER_TILE)])
        pltpu.sync_copy(eacc.at[pl.ds(base, ROWS_PER_TILE)],
                        out_hbm.at[cid, 1, pl.ds(base, ROWS_PER_TILE)])

    return hist


BN_ROWS = 1000


def _tc_matmul0(x, w):
    def body(x_ref, w_ref, o_ref):
        o_ref[...] = jnp.dot(x_ref[...], w_ref[...],
                             preferred_element_type=f32)

    d_in = x.shape[1]
    return pl.pallas_call(
        body,
        grid=(N // BN_ROWS,),
        in_specs=[
            pl.BlockSpec((BN_ROWS, d_in), lambda i: (i, 0)),
            pl.BlockSpec((d_in, HID), lambda i: (0, 0)),
        ],
        out_specs=pl.BlockSpec((BN_ROWS, HID), lambda i: (i, 0)),
        out_shape=jax.ShapeDtypeStruct((N, HID), f32),
    )(x, w)


def _tc_combine_e(pe, counts):
    def body(pe_ref, cnt_ref, o_ref):
        s = pe_ref[0] + pe_ref[1]
        edeg = cnt_ref[0, 0, :, 0:1] + cnt_ref[1, 0, :, 0:1]
        binv = jnp.where(edeg > 0, 1.0 / edeg, 0.0)
        o_ref[...] = s * binv

    return pl.pallas_call(
        body,
        grid=(M // BN_ROWS,),
        in_specs=[
            pl.BlockSpec((NC, BN_ROWS, HID), lambda i: (0, i, 0)),
            pl.BlockSpec((NC, 1, BN_ROWS, 16), lambda i: (0, 1, i, 0)),
        ],
        out_specs=pl.BlockSpec((BN_ROWS, HID), lambda i: (i, 0)),
        out_shape=jax.ShapeDtypeStruct((M, HID), f32),
    )(pe, counts)


def _tc_epilogue_matmul(pn, counts, b, g, be, rm, rv, w):
    def body(pn_ref, cnt_ref, b_ref, g_ref, be_ref, rm_ref, rv_ref, w_ref,
             o_ref):
        s = pn_ref[0] + pn_ref[1]
        deg = cnt_ref[0, 0, :, 0:1] + cnt_ref[1, 0, :, 0:1]
        dinv = jnp.where(deg > 0, 1.0 / deg, 0.0)
        scale = g_ref[...] * lax.rsqrt(rv_ref[...] + EPS)
        shift = (b_ref[...] - rm_ref[...]) * scale + be_ref[...]
        h = jnp.maximum(s * dinv * scale + shift, 0.0)
        o_ref[...] = jnp.dot(h, w_ref[...], preferred_element_type=f32)

    vec = lambda: pl.BlockSpec((1, HID), lambda i: (0, 0))
    return pl.pallas_call(
        body,
        grid=(N // BN_ROWS,),
        in_specs=[
            pl.BlockSpec((NC, BN_ROWS, HID), lambda i: (0, i, 0)),
            pl.BlockSpec((NC, 1, BN_ROWS, 16), lambda i: (0, 0, i, 0)),
            vec(), vec(), vec(), vec(), vec(),
            pl.BlockSpec((HID, HID), lambda i: (0, 0)),
        ],
        out_specs=pl.BlockSpec((BN_ROWS, HID), lambda i: (i, 0)),
        out_shape=jax.ShapeDtypeStruct((N, HID), f32),
    )(pn, counts, b.reshape(1, HID), g.reshape(1, HID), be.reshape(1, HID),
      rm.reshape(1, HID), rv.reshape(1, HID), w)


def _tc_epilogue_head(pn, counts, b, g, be, rm, rv, wc1, bc1, wc2, bc2):
    h1 = wc1.shape[1]
    ncls = wc2.shape[1]

    def body(pn_ref, cnt_ref, b_ref, g_ref, be_ref, rm_ref, rv_ref,
             wc1_ref, bc1_ref, wc2_ref, bc2_ref, o_ref):
        s = pn_ref[0] + pn_ref[1]
        deg = cnt_ref[0, 0, :, 0:1] + cnt_ref[1, 0, :, 0:1]
        dinv = jnp.where(deg > 0, 1.0 / deg, 0.0)
        scale = g_ref[...] * lax.rsqrt(rv_ref[...] + EPS)
        shift = (b_ref[...] - rm_ref[...]) * scale + be_ref[...]
        h = jnp.maximum(s * dinv * scale + shift, 0.0)
        t = jnp.maximum(
            jnp.dot(h, wc1_ref[...], preferred_element_type=f32)
            + bc1_ref[...], 0.0)
        o_ref[...] = (jnp.dot(t, wc2_ref[...], preferred_element_type=f32)
                      + bc2_ref[...])

    vec = lambda: pl.BlockSpec((1, HID), lambda i: (0, 0))
    return pl.pallas_call(
        body,
        grid=(N // BN_ROWS,),
        in_specs=[
            pl.BlockSpec((NC, BN_ROWS, HID), lambda i: (0, i, 0)),
            pl.BlockSpec((NC, 1, BN_ROWS, 16), lambda i: (0, 0, i, 0)),
            vec(), vec(), vec(), vec(), vec(),
            pl.BlockSpec((HID, h1), lambda i: (0, 0)),
            pl.BlockSpec((1, h1), lambda i: (0, 0)),
            pl.BlockSpec((h1, ncls), lambda i: (0, 0)),
            pl.BlockSpec((1, ncls), lambda i: (0, 0)),
        ],
        out_specs=pl.BlockSpec((BN_ROWS, ncls), lambda i: (i, 0)),
        out_shape=jax.ShapeDtypeStruct((N, ncls), f32),
    )(pn, counts, b.reshape(1, HID), g.reshape(1, HID), be.reshape(1, HID),
      rm.reshape(1, HID), rv.reshape(1, HID), wc1, bc1.reshape(1, h1),
      wc2, bc2.reshape(1, ncls))


def kernel(x, hyperedge_index, W0, b0, g0, be0, rm0, rv0,
           W1, b1, g1, be1, rm1, rv1, W2, b2, g2, be2, rm2, rv2,
           Wc1, bc1, Wc2, bc2):
    w_i = jnp.arange(NW, dtype=jnp.int32)[:, None]
    p_i = jnp.arange(PAD, dtype=jnp.int32)[None, :]
    pad_g = (w_i * 37 + p_i * 41) % N
    pad_s = N + (w_i + p_i) % (MP - N)

    def prep(idx, pad):
        return jnp.concatenate([idx.reshape(NW, PER_W), pad], axis=1).reshape(
            NW, NCHUNK, CHUNK)

    g_src = prep(hyperedge_index[0], pad_g)
    s_src = prep(hyperedge_index[0], pad_s)
    g_he = prep(hyperedge_index[1], pad_g)
    s_he = prep(hyperedge_index[1], pad_s)
    zeros64 = jnp.zeros((MP, HID), f32)
    zeros16 = jnp.zeros((MP, 16), f32)
    ones16 = jnp.ones((CHUNK, 16), f32)

    counts = _hist_call()(s_src, s_he, zeros16, ones16)
    stage = _stage_call()

    params = [(b0, g0, be0, rm0, rv0),
              (b1, g1, be1, rm1, rv1),
              (b2, g2, be2, rm2, rv2)]
    next_w = [W1, W2]
    xw = _tc_matmul0(x, W0)
    for li, (b, g, be, rm, rv) in enumerate(params):
        pe = stage(xw, g_src, s_he, zeros64)
        out_e = _tc_combine_e(pe, counts)
        pn = stage(out_e, g_he, s_src, zeros64)
        if li < 2:
            xw = _tc_epilogue_matmul(pn, counts, b, g, be, rm, rv, next_w[li])
        else:
            return _tc_epilogue_head(pn, counts, b, g, be, rm, rv,
                                     Wc1, bc1, Wc2, bc2)
